# baseline (speedup 1.0000x reference)
import functools

import jax
import jax.numpy as jnp
from jax import lax
from jax.experimental import pallas as pl
from jax.experimental.pallas import tpu as pltpu
from jax.experimental.pallas import tpu_sc as plsc

VOCAB = 1000000
EMBED_DIM = 64
BATCH = 16384
N_FIELDS = 26

NC, NS = 2, 16
NW = NC * NS
B = BATCH * N_FIELDS
B_PER_W = B // NW
IDX_W = 128
NROW = B_PER_W // IDX_W
GROUP = 512
GPG = GROUP // IDX_W
NGRP = B_PER_W // GROUP

_mesh = plsc.VectorSubcoreMesh(core_axis_name="c", subcore_axis_name="s")


@functools.partial(
    pl.kernel,
    out_type=jax.ShapeDtypeStruct((B, EMBED_DIM), jnp.float32),
    mesh=_mesh,
    scratch_types=[
        pltpu.VMEM((NROW, IDX_W), jnp.int32),
        pltpu.VMEM((GROUP, EMBED_DIM), jnp.float32),
        pltpu.SemaphoreType.DMA,
    ],
    compiler_params=pltpu.CompilerParams(use_tc_tiling_on_sc=False),
)
def _sc_gather(table, idx, out, idx_v, rows_v, gsem):
    wid = lax.axis_index("s") * NC + lax.axis_index("c")
    base = wid * B_PER_W
    pltpu.sync_copy(idx.at[wid], idx_v)

    def step(g):
        descs = [
            pltpu.async_copy(
                table.at[idx_v.at[g * GPG + j]],
                rows_v.at[pl.ds(j * IDX_W, IDX_W)],
                gsem,
            )
            for j in range(GPG)
        ]
        for d in descs:
            d.wait()
        pltpu.sync_copy(rows_v, out.at[pl.ds(base + g * GROUP, GROUP)])

    pl.loop(0, NGRP)(step)


def kernel(context, W):
    idx = context.astype(jnp.int32).reshape(NW, NROW, IDX_W)
    out = _sc_gather(W, idx)
    return out.reshape(BATCH, N_FIELDS, EMBED_DIM)


# speedup vs baseline: 1.0127x; 1.0127x over previous
import functools

import jax
import jax.numpy as jnp
from jax import lax
from jax.experimental import pallas as pl
from jax.experimental.pallas import tpu as pltpu
from jax.experimental.pallas import tpu_sc as plsc

VOCAB = 1000000
EMBED_DIM = 64
BATCH = 16384
N_FIELDS = 26

NC, NS = 2, 16
NW = NC * NS
B = BATCH * N_FIELDS
B_PER_W = B // NW
IDX_W = 128
NG = B_PER_W // IDX_W
NBUF = 8
NROUND = NG // NBUF

_mesh = plsc.VectorSubcoreMesh(core_axis_name="c", subcore_axis_name="s")

_scratch = (
    [pltpu.VMEM((NG, IDX_W), jnp.int32)]
    + [pltpu.VMEM((IDX_W, EMBED_DIM), jnp.float32) for _ in range(NBUF)]
    + [pltpu.SemaphoreType.DMA for _ in range(2 * NBUF)]
)


@functools.partial(
    pl.kernel,
    out_type=jax.ShapeDtypeStruct((B, EMBED_DIM), jnp.float32),
    mesh=_mesh,
    scratch_types=_scratch,
    compiler_params=pltpu.CompilerParams(use_tc_tiling_on_sc=False),
)
def _sc_gather(table, idx, out, idx_v, *bufs_and_sems):
    bufs = bufs_and_sems[:NBUF]
    gsems = bufs_and_sems[NBUF:2 * NBUF]
    ssems = bufs_and_sems[2 * NBUF:]
    wid = lax.axis_index("s") * NC + lax.axis_index("c")
    base = wid * B_PER_W
    pltpu.sync_copy(idx.at[wid], idx_v)

    def fire(g, b):
        pltpu.async_copy(table.at[idx_v.at[g]], bufs[b], gsems[b])

    def drain(b, sem):
        pltpu.make_async_copy(table.at[pl.ds(0, IDX_W)], bufs[b], sem).wait()

    for b in range(NBUF):
        fire(b, b)

    def step(i):
        for b in range(NBUF):
            g = i * NBUF + b
            drain(b, gsems[b])
            pltpu.async_copy(bufs[b], out.at[pl.ds(base + g * IDX_W, IDX_W)],
                             ssems[b])
        for b in range(NBUF):
            g_next = (i + 1) * NBUF + b

            @pl.when(g_next < NG)
            def _():
                drain(b, ssems[b])
                fire(g_next, b)

    pl.loop(0, NROUND)(step)
    for b in range(NBUF):
        drain(b, ssems[b])


def kernel(context, W):
    idx = context.astype(jnp.int32).reshape(NW, NG, IDX_W)
    out = _sc_gather(W, idx)
    return out.reshape(BATCH, N_FIELDS, EMBED_DIM)
